# Initial kernel scaffold; baseline (speedup 1.0000x reference)
import jax
import jax.numpy as jnp
from jax.experimental import pallas as pl


def kernel(input, dw_w, dw_b, ln_g, ln_b, off_w, off_b, mask_w, mask_b, in_w, in_b, out_w, out_b):
    raise NotImplementedError("write your pallas kernel here")


# speedup vs baseline: 138.3846x; 138.3846x over previous
import functools
import math

import jax
import jax.numpy as jnp
from jax import lax
from jax.experimental import pallas as pl
from jax.experimental.pallas import tpu as pltpu
from jax.experimental.pallas import tpu_sc as plsc

CH = 192
G = 12
GC = 16
P = 9
N, H, W = 4, 56, 56
HW = H * W
NPIX = N * HW
NTASK = NPIX * G

NWORKERS = 32
PIX_PER_W = NPIX // NWORKERS
CHUNK_PIX = 8
NCHUNK = PIX_PER_W // CHUNK_PIX
EPP = 4 * G * P
WPT = 112
CHUNK_E = CHUNK_PIX * G * P
CHUNK_ROWS = 4 * CHUNK_E
GATHER_B = 128
NGATHER = CHUNK_ROWS // GATHER_B


def _mm_body(x_ref, w_ref, b_ref, o_ref):
    o_ref[...] = jnp.dot(x_ref[...], w_ref[...],
                         preferred_element_type=jnp.float32) + b_ref[...]


def _matmul(x, wt, b, blocks=8):
    rows = x.shape[0]
    blk = rows // blocks
    cout = wt.shape[1]
    return pl.pallas_call(
        _mm_body,
        grid=(blocks,),
        in_specs=[
            pl.BlockSpec((blk, x.shape[1]), lambda i: (i, 0)),
            pl.BlockSpec((wt.shape[0], cout), lambda i: (0, 0)),
            pl.BlockSpec((1, cout), lambda i: (0, 0)),
        ],
        out_specs=pl.BlockSpec((blk, cout), lambda i: (i, 0)),
        out_shape=jax.ShapeDtypeStruct((rows, cout), jnp.float32),
    )(x, wt, b)


def _prep_body(x_ref, dwt_ref, lng_ref, lnb_ref, owx_ref, obx_ref, owy_ref,
               oby_ref, mwt_ref, mb_ref, S_ref, lkx_ref, lky_ref, gofl_ref,
               i0_ref, i1_ref, i2_ref, i3_ref, w0_ref, w1_ref, w2_ref, w3_ref):
    n = pl.program_id(0)
    x3 = x_ref[0]
    zr = jnp.zeros((1, W, CH), jnp.float32)
    zc = jnp.zeros((H + 2, 1, CH), jnp.float32)
    xp2 = jnp.concatenate([zr, x3, zr], axis=0)
    xp2 = jnp.concatenate([zc, xp2, zc], axis=1)
    x1 = jnp.zeros((H, W, CH), jnp.float32)
    for dy in range(3):
        for dx in range(3):
            x1 = x1 + xp2[dy:dy + H, dx:dx + W, :] * dwt_ref[dy, dx, :]
    x1 = x1.reshape(HW, CH)
    mu = jnp.mean(x1, axis=-1, keepdims=True)
    xc = x1 - mu
    var = jnp.mean(xc * xc, axis=-1, keepdims=True)
    x1 = xc * lax.rsqrt(var + 1e-6) * lng_ref[...] + lnb_ref[...]
    x1 = 0.5 * x1 * (1.0 + lax.erf(x1 * jnp.float32(1.0 / math.sqrt(2.0))))

    offx = jnp.dot(x1, owx_ref[...], preferred_element_type=jnp.float32) + obx_ref[...]
    offy = jnp.dot(x1, owy_ref[...], preferred_element_type=jnp.float32) + oby_ref[...]
    m = jnp.dot(x1, mwt_ref[...], preferred_element_type=jnp.float32) + mb_ref[...]
    em = jnp.exp(m)
    den = jnp.dot(em, S_ref[...], preferred_element_type=jnp.float32)
    msm = em / den

    pix = lax.broadcasted_iota(jnp.int32, (HW, G * P), 0)
    oy = pix // W
    ox = pix - oy * W
    gx = (ox + 1).astype(jnp.float32) + lkx_ref[...] + offx
    gy = (oy + 1).astype(jnp.float32) + lky_ref[...] + offy
    x0 = jnp.floor(gx)
    y0 = jnp.floor(gy)
    wx1 = gx - x0
    wx0 = 1.0 - wx1
    wy1 = gy - y0
    wy0 = 1.0 - wy1
    rowbase = n * (HW * G)

    outs = ((i0_ref, w0_ref, 0, 0, wx0 * wy0), (i1_ref, w1_ref, 1, 0, wx1 * wy0),
            (i2_ref, w2_ref, 0, 1, wx0 * wy1), (i3_ref, w3_ref, 1, 1, wx1 * wy1))
    for i_ref, w_ref, tx_off, ty_off, wt in outs:
        tx = x0 + float(tx_off)
        ty = y0 + float(ty_off)
        ok = (tx >= 1.0) & (tx <= 56.0) & (ty >= 1.0) & (ty <= 56.0)
        pxu = jnp.clip(tx, 1.0, 56.0).astype(jnp.int32) - 1
        pyu = jnp.clip(ty, 1.0, 56.0).astype(jnp.int32) - 1
        i_ref[...] = (rowbase + (pyu * W + pxu) * G) + gofl_ref[...]
        w_ref[:, pl.ds(0, G * P)] = wt * msm * ok.astype(jnp.float32)
        w_ref[:, pl.ds(G * P, WPT - G * P)] = jnp.zeros((HW, WPT - G * P),
                                                        jnp.float32)


def _prep(x, dwt, lng, lnb, owx, obx, owy, oby, mwt, mb, S, lkx, lky, gofl):
    full = lambda shp: pl.BlockSpec(shp, lambda n: tuple(0 for _ in shp))
    io_spec = pl.BlockSpec((HW, G * P), lambda n: (n, 0))
    wo_spec = pl.BlockSpec((HW, WPT), lambda n: (n, 0))
    out_shape = [jax.ShapeDtypeStruct((NPIX, G * P), jnp.int32)] * 4 + \
                [jax.ShapeDtypeStruct((NPIX, WPT), jnp.float32)] * 4
    return pl.pallas_call(
        _prep_body,
        grid=(N,),
        in_specs=[
            pl.BlockSpec((1, H, W, CH), lambda n: (n, 0, 0, 0)),
            full((3, 3, CH)), full((1, CH)), full((1, CH)),
            full((CH, G * P)), full((1, G * P)),
            full((CH, G * P)), full((1, G * P)),
            full((CH, G * P)), full((1, G * P)),
            full((G * P, G * P)),
            full((1, G * P)), full((1, G * P)), full((1, G * P)),
        ],
        out_specs=[io_spec] * 4 + [wo_spec] * 4,
        out_shape=out_shape,
    )(x, dwt, lng, lnb, owx, obx, owy, oby, mwt, mb, S, lkx, lky, gofl)


def _sc_body(table, i0, i1, i2, i3, w0, w1, w2, w3, y,
             idx_v, w_v, rows_v, out_v, sem):
    wid = lax.axis_index("s") * 2 + lax.axis_index("c")

    def chunk_body(ci, _):
        pixbase = wid * PIX_PER_W + ci * CHUNK_PIX
        ebase = pixbase * (G * P)
        for t, r in enumerate((i0, i1, i2, i3)):
            pltpu.sync_copy(r.at[pl.ds(ebase, CHUNK_E)],
                            idx_v.at[pl.ds(t * CHUNK_E, CHUNK_E)])
        for t, r in enumerate((w0, w1, w2, w3)):
            pltpu.sync_copy(r.at[pl.ds(pixbase, CHUNK_PIX), :],
                            w_v.at[:, pl.ds(t * WPT, WPT)])
        cps = [pltpu.async_copy(
                   table.at[idx_v.at[pl.ds(k * GATHER_B, GATHER_B)]],
                   rows_v.at[pl.ds(k * GATHER_B, GATHER_B)], sem)
               for k in range(NGATHER)]
        for cp in cps:
            cp.wait()

        def pix_body(pj, _):
            accs = [jnp.zeros((GC,), jnp.float32) for _ in range(G)]
            for t in range(4):
                for k in range(WPT // GC):
                    wv = w_v[pj, pl.ds(t * WPT + k * GC, GC)]
                    for j in range(GC):
                        l = k * GC + j
                        if l >= G * P:
                            continue
                        accs[l // P] = accs[l // P] + \
                            rows_v[t * CHUNK_E + pj * (G * P) + l, :] * wv[j]
            for g in range(G):
                out_v[pj * G + g, :] = accs[g]
            return 0

        lax.fori_loop(0, CHUNK_PIX, pix_body, 0)
        pltpu.sync_copy(out_v, y.at[pl.ds(pixbase * G, CHUNK_PIX * G)])
        return 0

    lax.fori_loop(0, NCHUNK, chunk_body, 0)


@functools.lru_cache(maxsize=1)
def _sc_gather():
    return pl.kernel(
        _sc_body,
        mesh=plsc.VectorSubcoreMesh(core_axis_name="c", subcore_axis_name="s"),
        compiler_params=pltpu.CompilerParams(use_tc_tiling_on_sc=False),
        out_type=jax.ShapeDtypeStruct((NTASK, GC), jnp.float32),
        scratch_types=[
            pltpu.VMEM((CHUNK_ROWS,), jnp.int32),
            pltpu.VMEM((CHUNK_PIX, 4 * WPT), jnp.float32),
            pltpu.VMEM((CHUNK_ROWS, GC), jnp.float32),
            pltpu.VMEM((CHUNK_PIX * G, GC), jnp.float32),
            pltpu.SemaphoreType.DMA,
        ],
    )


def kernel(input, dw_w, dw_b, ln_g, ln_b, off_w, off_b, mask_w, mask_b,
           in_w, in_b, out_w, out_b):
    f32 = jnp.float32
    x = input.astype(f32)

    xp = _matmul(x.reshape(NPIX, CH), in_w.T, in_b.reshape(1, CH))
    table = xp.reshape(NTASK, GC)

    dwt = jnp.transpose(dw_w[:, 0, :, :], (1, 2, 0))
    owx = off_w[0::2].T
    owy = off_w[1::2].T
    obx = off_b[0::2].reshape(1, G * P)
    oby = off_b[1::2].reshape(1, G * P)
    S = jnp.kron(jnp.eye(G, dtype=f32), jnp.ones((P, P), f32))
    lk = jnp.array([-1.0, 0.0, 1.0], f32)
    lidx = jnp.arange(G * P)
    lkx = lk[(lidx % P) // 3].reshape(1, G * P)
    lky = lk[(lidx % P) % 3].reshape(1, G * P)
    gofl = (lidx // P).astype(jnp.int32).reshape(1, G * P)

    i0, i1, i2, i3, w0, w1, w2, w3 = _prep(
        x, dwt, ln_g.reshape(1, CH), ln_b.reshape(1, CH),
        owx, obx, owy, oby, mask_w.T, mask_b.reshape(1, G * P),
        S, lkx, lky, gofl)

    flat = lambda a: a.reshape(NPIX * G * P)
    y = _sc_gather()(table, flat(i0), flat(i1), flat(i2), flat(i3),
                     w0, w1, w2, w3)

    out = _matmul(y.reshape(NPIX, CH), out_w.T, out_b.reshape(1, CH))
    return out.reshape(N, H, W, CH)


# speedup vs baseline: 158.5964x; 1.1461x over previous
import functools
import math

import jax
import jax.numpy as jnp
from jax import lax
from jax.experimental import pallas as pl
from jax.experimental.pallas import tpu as pltpu
from jax.experimental.pallas import tpu_sc as plsc

CH = 192
G = 12
GC = 16
P = 9
N, H, W = 4, 56, 56
HW = H * W
NPIX = N * HW
NTASK = NPIX * G

NWORKERS = 32
PIX_PER_W = NPIX // NWORKERS
CHUNK_PIX = 8
NCHUNK = PIX_PER_W // CHUNK_PIX
EPP = 4 * G * P
WPT = 112
CHUNK_E = CHUNK_PIX * G * P
CHUNK_ROWS = 4 * CHUNK_E
GATHER_B = 128
NGATHER = CHUNK_ROWS // GATHER_B


def _mm_body(x_ref, w_ref, b_ref, o_ref):
    o_ref[...] = jnp.dot(x_ref[...], w_ref[...],
                         preferred_element_type=jnp.float32) + b_ref[...]


def _matmul(x, wt, b, blocks=8):
    rows = x.shape[0]
    blk = rows // blocks
    cout = wt.shape[1]
    return pl.pallas_call(
        _mm_body,
        grid=(blocks,),
        in_specs=[
            pl.BlockSpec((blk, x.shape[1]), lambda i: (i, 0)),
            pl.BlockSpec((wt.shape[0], cout), lambda i: (0, 0)),
            pl.BlockSpec((1, cout), lambda i: (0, 0)),
        ],
        out_specs=pl.BlockSpec((blk, cout), lambda i: (i, 0)),
        out_shape=jax.ShapeDtypeStruct((rows, cout), jnp.float32),
    )(x, wt, b)


def _prep_body(x_ref, dwt_ref, lng_ref, lnb_ref, owx_ref, obx_ref, owy_ref,
               oby_ref, mwt_ref, mb_ref, S_ref, lkx_ref, lky_ref, gofl_ref,
               i0_ref, i1_ref, i2_ref, i3_ref, w0_ref, w1_ref, w2_ref, w3_ref):
    n = pl.program_id(0)
    x3 = x_ref[0]
    zr = jnp.zeros((1, W, CH), jnp.float32)
    zc = jnp.zeros((H + 2, 1, CH), jnp.float32)
    xp2 = jnp.concatenate([zr, x3, zr], axis=0)
    xp2 = jnp.concatenate([zc, xp2, zc], axis=1)
    x1 = jnp.zeros((H, W, CH), jnp.float32)
    for dy in range(3):
        for dx in range(3):
            x1 = x1 + xp2[dy:dy + H, dx:dx + W, :] * dwt_ref[dy, dx, :]
    x1 = x1.reshape(HW, CH)
    mu = jnp.mean(x1, axis=-1, keepdims=True)
    xc = x1 - mu
    var = jnp.mean(xc * xc, axis=-1, keepdims=True)
    x1 = xc * lax.rsqrt(var + 1e-6) * lng_ref[...] + lnb_ref[...]
    x1 = 0.5 * x1 * (1.0 + lax.erf(x1 * jnp.float32(1.0 / math.sqrt(2.0))))

    offx = jnp.dot(x1, owx_ref[...], preferred_element_type=jnp.float32) + obx_ref[...]
    offy = jnp.dot(x1, owy_ref[...], preferred_element_type=jnp.float32) + oby_ref[...]
    m = jnp.dot(x1, mwt_ref[...], preferred_element_type=jnp.float32) + mb_ref[...]
    em = jnp.exp(m)
    den = jnp.dot(em, S_ref[...], preferred_element_type=jnp.float32)
    msm = em / den

    pix = lax.broadcasted_iota(jnp.int32, (HW, G * P), 0)
    oy = pix // W
    ox = pix - oy * W
    gx = (ox + 1).astype(jnp.float32) + lkx_ref[...] + offx
    gy = (oy + 1).astype(jnp.float32) + lky_ref[...] + offy
    x0 = jnp.floor(gx)
    y0 = jnp.floor(gy)
    wx1 = gx - x0
    wx0 = 1.0 - wx1
    wy1 = gy - y0
    wy0 = 1.0 - wy1
    rowbase = n * (HW * G)

    outs = ((i0_ref, w0_ref, 0, 0, wx0 * wy0), (i1_ref, w1_ref, 1, 0, wx1 * wy0),
            (i2_ref, w2_ref, 0, 1, wx0 * wy1), (i3_ref, w3_ref, 1, 1, wx1 * wy1))
    for i_ref, w_ref, tx_off, ty_off, wt in outs:
        tx = x0 + float(tx_off)
        ty = y0 + float(ty_off)
        ok = (tx >= 1.0) & (tx <= 56.0) & (ty >= 1.0) & (ty <= 56.0)
        pxu = jnp.clip(tx, 1.0, 56.0).astype(jnp.int32) - 1
        pyu = jnp.clip(ty, 1.0, 56.0).astype(jnp.int32) - 1
        i_ref[...] = (rowbase + (pyu * W + pxu) * G) + gofl_ref[...]
        w_ref[:, pl.ds(0, G * P)] = wt * msm * ok.astype(jnp.float32)
        w_ref[:, pl.ds(G * P, WPT - G * P)] = jnp.zeros((HW, WPT - G * P),
                                                        jnp.float32)


def _prep(x, dwt, lng, lnb, owx, obx, owy, oby, mwt, mb, S, lkx, lky, gofl):
    full = lambda shp: pl.BlockSpec(shp, lambda n: tuple(0 for _ in shp))
    io_spec = pl.BlockSpec((HW, G * P), lambda n: (n, 0))
    wo_spec = pl.BlockSpec((HW, WPT), lambda n: (n, 0))
    out_shape = [jax.ShapeDtypeStruct((NPIX, G * P), jnp.int32)] * 4 + \
                [jax.ShapeDtypeStruct((NPIX, WPT), jnp.float32)] * 4
    return pl.pallas_call(
        _prep_body,
        grid=(N,),
        in_specs=[
            pl.BlockSpec((1, H, W, CH), lambda n: (n, 0, 0, 0)),
            full((3, 3, CH)), full((1, CH)), full((1, CH)),
            full((CH, G * P)), full((1, G * P)),
            full((CH, G * P)), full((1, G * P)),
            full((CH, G * P)), full((1, G * P)),
            full((G * P, G * P)),
            full((1, G * P)), full((1, G * P)), full((1, G * P)),
        ],
        out_specs=[io_spec] * 4 + [wo_spec] * 4,
        out_shape=out_shape,
    )(x, dwt, lng, lnb, owx, obx, owy, oby, mwt, mb, S, lkx, lky, gofl)


def _sc_body(table, i0, i1, i2, i3, w0, w1, w2, w3, y,
             idx_v, w_v, rows_v, out_v, sem):
    wid = lax.axis_index("s") * 2 + lax.axis_index("c")

    def stage(buf, ci):
        pixbase = wid * PIX_PER_W + ci * CHUNK_PIX
        ebase = pixbase * (G * P)
        for t, r in enumerate((i0, i1, i2, i3)):
            pltpu.sync_copy(r.at[pl.ds(ebase, CHUNK_E)],
                            idx_v.at[buf, pl.ds(t * CHUNK_E, CHUNK_E)])
        for t, r in enumerate((w0, w1, w2, w3)):
            pltpu.sync_copy(r.at[pl.ds(pixbase, CHUNK_PIX), :],
                            w_v.at[buf, :, pl.ds(t * WPT, WPT)])
        for k in range(NGATHER):
            pltpu.async_copy(
                table.at[idx_v.at[buf, pl.ds(k * GATHER_B, GATHER_B)]],
                rows_v.at[buf, pl.ds(k * GATHER_B, GATHER_B)], sem)

    stage(0, 0)

    def chunk_body(ci, _):
        buf = lax.rem(ci, 2)
        pixbase = wid * PIX_PER_W + ci * CHUNK_PIX
        for k in range(NGATHER):
            pltpu.make_async_copy(
                table.at[idx_v.at[buf, pl.ds(k * GATHER_B, GATHER_B)]],
                rows_v.at[buf, pl.ds(k * GATHER_B, GATHER_B)], sem).wait()

        @pl.when(ci + 1 < NCHUNK)
        def _():
            stage(1 - buf, ci + 1)

        def pix_body(pj, _):
            accs = [jnp.zeros((GC,), jnp.float32) for _ in range(G)]
            for t in range(4):
                for k in range(WPT // GC):
                    wv = w_v[buf, pj, pl.ds(t * WPT + k * GC, GC)]
                    for j in range(GC):
                        l = k * GC + j
                        if l >= G * P:
                            continue
                        accs[l // P] = accs[l // P] + \
                            rows_v[buf, t * CHUNK_E + pj * (G * P) + l, :] * wv[j]
            for g in range(G):
                out_v[pj * G + g, :] = accs[g]
            return 0

        lax.fori_loop(0, CHUNK_PIX, pix_body, 0)
        pltpu.sync_copy(out_v, y.at[pl.ds(pixbase * G, CHUNK_PIX * G)])
        return 0

    lax.fori_loop(0, NCHUNK, chunk_body, 0)


@functools.lru_cache(maxsize=1)
def _sc_gather():
    return pl.kernel(
        _sc_body,
        mesh=plsc.VectorSubcoreMesh(core_axis_name="c", subcore_axis_name="s"),
        compiler_params=pltpu.CompilerParams(use_tc_tiling_on_sc=False),
        out_type=jax.ShapeDtypeStruct((NTASK, GC), jnp.float32),
        scratch_types=[
            pltpu.VMEM((2, CHUNK_ROWS), jnp.int32),
            pltpu.VMEM((2, CHUNK_PIX, 4 * WPT), jnp.float32),
            pltpu.VMEM((2, CHUNK_ROWS, GC), jnp.float32),
            pltpu.VMEM((CHUNK_PIX * G, GC), jnp.float32),
            pltpu.SemaphoreType.DMA,
        ],
    )


def kernel(input, dw_w, dw_b, ln_g, ln_b, off_w, off_b, mask_w, mask_b,
           in_w, in_b, out_w, out_b):
    f32 = jnp.float32
    x = input.astype(f32)

    xp = _matmul(x.reshape(NPIX, CH), in_w.T, in_b.reshape(1, CH))
    table = xp.reshape(NTASK, GC)

    dwt = jnp.transpose(dw_w[:, 0, :, :], (1, 2, 0))
    owx = off_w[0::2].T
    owy = off_w[1::2].T
    obx = off_b[0::2].reshape(1, G * P)
    oby = off_b[1::2].reshape(1, G * P)
    S = jnp.kron(jnp.eye(G, dtype=f32), jnp.ones((P, P), f32))
    lk = jnp.array([-1.0, 0.0, 1.0], f32)
    lidx = jnp.arange(G * P)
    lkx = lk[(lidx % P) // 3].reshape(1, G * P)
    lky = lk[(lidx % P) % 3].reshape(1, G * P)
    gofl = (lidx // P).astype(jnp.int32).reshape(1, G * P)

    i0, i1, i2, i3, w0, w1, w2, w3 = _prep(
        x, dwt, ln_g.reshape(1, CH), ln_b.reshape(1, CH),
        owx, obx, owy, oby, mask_w.T, mask_b.reshape(1, G * P),
        S, lkx, lky, gofl)

    flat = lambda a: a.reshape(NPIX * G * P)
    y = _sc_gather()(table, flat(i0), flat(i1), flat(i2), flat(i3),
                     w0, w1, w2, w3)

    out = _matmul(y.reshape(NPIX, CH), out_w.T, out_b.reshape(1, CH))
    return out.reshape(N, H, W, CH)


# speedup vs baseline: 227.1808x; 1.4324x over previous
import functools
import math

import jax
import jax.numpy as jnp
from jax import lax
from jax.experimental import pallas as pl
from jax.experimental.pallas import tpu as pltpu
from jax.experimental.pallas import tpu_sc as plsc

CH = 192
G = 12
GC = 16
P = 9
N, H, W = 4, 56, 56
HW = H * W
NPIX = N * HW
NTASK = NPIX * G

NWORKERS = 32
PIX_PER_W = NPIX // NWORKERS
CHUNK_PIX = 8
NCHUNK = PIX_PER_W // CHUNK_PIX
EPP = 4 * G * P
WPT = 112
CHUNK_E = CHUNK_PIX * G * P
CHUNK_ROWS = 4 * CHUNK_E
GATHER_B = 128
NGATHER = CHUNK_ROWS // GATHER_B


def _mm_body(x_ref, w_ref, b_ref, o_ref):
    o_ref[...] = jnp.dot(x_ref[...], w_ref[...],
                         preferred_element_type=jnp.float32) + b_ref[...]


def _matmul(x, wt, b, blocks=8):
    rows = x.shape[0]
    blk = rows // blocks
    cout = wt.shape[1]
    return pl.pallas_call(
        _mm_body,
        grid=(blocks,),
        in_specs=[
            pl.BlockSpec((blk, x.shape[1]), lambda i: (i, 0)),
            pl.BlockSpec((wt.shape[0], cout), lambda i: (0, 0)),
            pl.BlockSpec((1, cout), lambda i: (0, 0)),
        ],
        out_specs=pl.BlockSpec((blk, cout), lambda i: (i, 0)),
        out_shape=jax.ShapeDtypeStruct((rows, cout), jnp.float32),
    )(x, wt, b)


def _prep_body(x_ref, dwt_ref, lng_ref, lnb_ref, owx_ref, obx_ref, owy_ref,
               oby_ref, mwt_ref, mb_ref, S_ref, lkx_ref, lky_ref, gofl_ref,
               i0_ref, i1_ref, i2_ref, i3_ref, w0_ref, w1_ref, w2_ref, w3_ref):
    n = pl.program_id(0)
    x3 = x_ref[0]
    zr = jnp.zeros((1, W, CH), jnp.float32)
    zc = jnp.zeros((H + 2, 1, CH), jnp.float32)
    xp2 = jnp.concatenate([zr, x3, zr], axis=0)
    xp2 = jnp.concatenate([zc, xp2, zc], axis=1)
    x1 = jnp.zeros((H, W, CH), jnp.float32)
    for dy in range(3):
        for dx in range(3):
            x1 = x1 + xp2[dy:dy + H, dx:dx + W, :] * dwt_ref[dy, dx, :]
    x1 = x1.reshape(HW, CH)
    mu = jnp.mean(x1, axis=-1, keepdims=True)
    xc = x1 - mu
    var = jnp.mean(xc * xc, axis=-1, keepdims=True)
    x1 = xc * lax.rsqrt(var + 1e-6) * lng_ref[...] + lnb_ref[...]
    x1 = 0.5 * x1 * (1.0 + lax.erf(x1 * jnp.float32(1.0 / math.sqrt(2.0))))

    offx = jnp.dot(x1, owx_ref[...], preferred_element_type=jnp.float32) + obx_ref[...]
    offy = jnp.dot(x1, owy_ref[...], preferred_element_type=jnp.float32) + oby_ref[...]
    m = jnp.dot(x1, mwt_ref[...], preferred_element_type=jnp.float32) + mb_ref[...]
    em = jnp.exp(m)
    den = jnp.dot(em, S_ref[...], preferred_element_type=jnp.float32)
    msm = em / den

    pix = lax.broadcasted_iota(jnp.int32, (HW, G * P), 0)
    oy = pix // W
    ox = pix - oy * W
    gx = (ox + 1).astype(jnp.float32) + lkx_ref[...] + offx
    gy = (oy + 1).astype(jnp.float32) + lky_ref[...] + offy
    x0 = jnp.floor(gx)
    y0 = jnp.floor(gy)
    wx1 = gx - x0
    wx0 = 1.0 - wx1
    wy1 = gy - y0
    wy0 = 1.0 - wy1
    rowbase = n * (HW * G)

    outs = ((i0_ref, w0_ref, 0, 0, wx0 * wy0), (i1_ref, w1_ref, 1, 0, wx1 * wy0),
            (i2_ref, w2_ref, 0, 1, wx0 * wy1), (i3_ref, w3_ref, 1, 1, wx1 * wy1))
    for i_ref, w_ref, tx_off, ty_off, wt in outs:
        tx = x0 + float(tx_off)
        ty = y0 + float(ty_off)
        ok = (tx >= 1.0) & (tx <= 56.0) & (ty >= 1.0) & (ty <= 56.0)
        pxu = jnp.clip(tx, 1.0, 56.0).astype(jnp.int32) - 1
        pyu = jnp.clip(ty, 1.0, 56.0).astype(jnp.int32) - 1
        i_ref[...] = (rowbase + (pyu * W + pxu) * G) + gofl_ref[...]
        w_ref[:, pl.ds(0, G * P)] = wt * msm * ok.astype(jnp.float32)
        w_ref[:, pl.ds(G * P, WPT - G * P)] = jnp.zeros((HW, WPT - G * P),
                                                        jnp.float32)


def _prep(x, dwt, lng, lnb, owx, obx, owy, oby, mwt, mb, S, lkx, lky, gofl):
    full = lambda shp: pl.BlockSpec(shp, lambda n: tuple(0 for _ in shp))
    io_spec = pl.BlockSpec((HW, G * P), lambda n: (n, 0))
    wo_spec = pl.BlockSpec((HW, WPT), lambda n: (n, 0))
    out_shape = [jax.ShapeDtypeStruct((NPIX, G * P), jnp.int32)] * 4 + \
                [jax.ShapeDtypeStruct((NPIX, WPT), jnp.float32)] * 4
    return pl.pallas_call(
        _prep_body,
        grid=(N,),
        in_specs=[
            pl.BlockSpec((1, H, W, CH), lambda n: (n, 0, 0, 0)),
            full((3, 3, CH)), full((1, CH)), full((1, CH)),
            full((CH, G * P)), full((1, G * P)),
            full((CH, G * P)), full((1, G * P)),
            full((CH, G * P)), full((1, G * P)),
            full((G * P, G * P)),
            full((1, G * P)), full((1, G * P)), full((1, G * P)),
        ],
        out_specs=[io_spec] * 4 + [wo_spec] * 4,
        out_shape=out_shape,
    )(x, dwt, lng, lnb, owx, obx, owy, oby, mwt, mb, S, lkx, lky, gofl)


def _sc_body(table, i0, i1, i2, i3, w0, w1, w2, w3, y,
             idx_v, w_v, rows_v, out_v, sem_i, sem_w, sem_g):
    wid = lax.axis_index("s") * 2 + lax.axis_index("c")

    def idx_copies(buf, ci):
        pixbase = wid * PIX_PER_W + ci * CHUNK_PIX
        ebase = pixbase * (G * P)
        return [pltpu.make_async_copy(r.at[pl.ds(ebase, CHUNK_E)],
                                      idx_v.at[buf, pl.ds(t * CHUNK_E, CHUNK_E)],
                                      sem_i)
                for t, r in enumerate((i0, i1, i2, i3))]

    def w_copies(buf, ci):
        pixbase = wid * PIX_PER_W + ci * CHUNK_PIX
        return [pltpu.make_async_copy(r.at[pl.ds(pixbase, CHUNK_PIX), :],
                                      w_v.at[buf, :, pl.ds(t * WPT, WPT)],
                                      sem_w)
                for t, r in enumerate((w0, w1, w2, w3))]

    def gather_copies(buf):
        return [pltpu.make_async_copy(
                    table.at[idx_v.at[buf, pl.ds(k * GATHER_B, GATHER_B)]],
                    rows_v.at[buf, pl.ds(k * GATHER_B, GATHER_B)], sem_g)
                for k in range(NGATHER)]

    for cp in idx_copies(0, 0) + w_copies(0, 0):
        cp.start()
        cp.wait()
    for cp in gather_copies(0):
        cp.start()
    for cp in idx_copies(1, 1) + w_copies(1, 1):
        cp.start()

    def chunk_body(ci, _):
        buf = lax.rem(ci, 2)
        nbuf = 1 - buf
        pixbase = wid * PIX_PER_W + ci * CHUNK_PIX

        @pl.when(ci + 1 < NCHUNK)
        def _():
            for cp in idx_copies(nbuf, ci + 1) + w_copies(nbuf, ci + 1):
                cp.wait()
            for cp in gather_copies(nbuf):
                cp.start()

        for cp in gather_copies(buf):
            cp.wait()

        @pl.when(ci + 2 < NCHUNK)
        def _():
            for cp in idx_copies(buf, ci + 2):
                cp.start()

        def pix_body(pj, _):
            accs = [jnp.zeros((GC,), jnp.float32) for _ in range(G)]
            for t in range(4):
                wvs = [w_v[buf, pj, pl.ds(t * WPT + k * GC, GC)]
                       for k in range(WPT // GC)]
                for p in range(P):
                    for g in range(G):
                        l = g * P + p
                        accs[g] = accs[g] + \
                            rows_v[buf, t * CHUNK_E + pj * (G * P) + l, :] * \
                            wvs[l // GC][l % GC]
            for g in range(G):
                out_v[pj * G + g, :] = accs[g]
            return 0

        lax.fori_loop(0, CHUNK_PIX, pix_body, 0)

        @pl.when(ci + 2 < NCHUNK)
        def _():
            for cp in w_copies(buf, ci + 2):
                cp.start()

        pltpu.sync_copy(out_v, y.at[pl.ds(pixbase * G, CHUNK_PIX * G)])
        return 0

    lax.fori_loop(0, NCHUNK, chunk_body, 0)


@functools.lru_cache(maxsize=1)
def _sc_gather():
    return pl.kernel(
        _sc_body,
        mesh=plsc.VectorSubcoreMesh(core_axis_name="c", subcore_axis_name="s"),
        compiler_params=pltpu.CompilerParams(use_tc_tiling_on_sc=False),
        out_type=jax.ShapeDtypeStruct((NTASK, GC), jnp.float32),
        scratch_types=[
            pltpu.VMEM((2, CHUNK_ROWS), jnp.int32),
            pltpu.VMEM((2, CHUNK_PIX, 4 * WPT), jnp.float32),
            pltpu.VMEM((2, CHUNK_ROWS, GC), jnp.float32),
            pltpu.VMEM((CHUNK_PIX * G, GC), jnp.float32),
            pltpu.SemaphoreType.DMA,
            pltpu.SemaphoreType.DMA,
            pltpu.SemaphoreType.DMA,
        ],
    )


def kernel(input, dw_w, dw_b, ln_g, ln_b, off_w, off_b, mask_w, mask_b,
           in_w, in_b, out_w, out_b):
    f32 = jnp.float32
    x = input.astype(f32)

    xp = _matmul(x.reshape(NPIX, CH), in_w.T, in_b.reshape(1, CH))
    table = xp.reshape(NTASK, GC)

    dwt = jnp.transpose(dw_w[:, 0, :, :], (1, 2, 0))
    owx = off_w[0::2].T
    owy = off_w[1::2].T
    obx = off_b[0::2].reshape(1, G * P)
    oby = off_b[1::2].reshape(1, G * P)
    S = jnp.kron(jnp.eye(G, dtype=f32), jnp.ones((P, P), f32))
    lk = jnp.array([-1.0, 0.0, 1.0], f32)
    lidx = jnp.arange(G * P)
    lkx = lk[(lidx % P) // 3].reshape(1, G * P)
    lky = lk[(lidx % P) % 3].reshape(1, G * P)
    gofl = (lidx // P).astype(jnp.int32).reshape(1, G * P)

    i0, i1, i2, i3, w0, w1, w2, w3 = _prep(
        x, dwt, ln_g.reshape(1, CH), ln_b.reshape(1, CH),
        owx, obx, owy, oby, mask_w.T, mask_b.reshape(1, G * P),
        S, lkx, lky, gofl)

    flat = lambda a: a.reshape(NPIX * G * P)
    y = _sc_gather()(table, flat(i0), flat(i1), flat(i2), flat(i3),
                     w0, w1, w2, w3)

    out = _matmul(y.reshape(NPIX, CH), out_w.T, out_b.reshape(1, CH))
    return out.reshape(N, H, W, CH)
